# baseline (speedup 1.0000x reference)
import jax
import jax.numpy as jnp
from jax.experimental import pallas as pl
from jax.experimental.pallas import tpu as pltpu
from jax.experimental.pallas import tpu_sc as plsc

_VMESH = plsc.VectorSubcoreMesh(core_axis_name="c", subcore_axis_name="s")
_SBT = 16
_LANES = 16


def _sc_body(x_vmem, pe_vmem, o_vmem):
    rows, cols = x_vmem.shape

    @pl.loop(0, rows)
    def _(r):
        @pl.loop(0, cols, step=_LANES)
        def _(c):
            slc = (pl.ds(r, 1), pl.ds(c, _LANES))
            o_vmem.at[*slc][...] = x_vmem.at[*slc][...] + pe_vmem.at[*slc][...]


def _sc_add(x2d, pe):
    n_rows, D = x2d.shape
    T = pe.shape[0]
    nblk = n_rows // _SBT
    tblk = T // _SBT

    @pl.kernel(out_type=jax.ShapeDtypeStruct(x2d.shape, x2d.dtype), mesh=_VMESH)
    def k(x_hbm, pe_hbm, o_hbm):
        pltpu.emit_pipeline(
            _sc_body,
            grid=(nblk,),
            in_specs=[
                pl.BlockSpec((_SBT, D), lambda i: (i, 0)),
                pl.BlockSpec((_SBT, D), lambda i: (i % tblk, 0)),
            ],
            out_specs=[pl.BlockSpec((_SBT, D), lambda i: (i, 0))],
            core_axis_name=("c", "s"),
            dimension_semantics=(pltpu.PARALLEL,),
        )(x_hbm, pe_hbm, o_hbm)

    return k(x2d, pe)


def kernel(x, pos_emb):
    B, T, D = x.shape
    pe = pos_emb[:T]
    out2d = _sc_add(x.reshape(B * T, D), pe)
    return out2d.reshape(B, T, D)


# speedup vs baseline: 4.3161x; 4.3161x over previous
import jax
import jax.numpy as jnp
from jax.experimental import pallas as pl
from jax.experimental.pallas import tpu as pltpu

BT = 512


def _add_body(x_ref, pe_ref, o_ref):
    o_ref[...] = x_ref[...] + pe_ref[...][None, :, :]


def kernel(x, pos_emb):
    B, T, D = x.shape
    pe = pos_emb[:T]
    grid = (T // BT,)
    return pl.pallas_call(
        _add_body,
        grid=grid,
        in_specs=[
            pl.BlockSpec((B, BT, D), lambda i: (0, i, 0)),
            pl.BlockSpec((BT, D), lambda i: (i, 0)),
        ],
        out_specs=pl.BlockSpec((B, BT, D), lambda i: (0, i, 0)),
        out_shape=jax.ShapeDtypeStruct((B, T, D), x.dtype),
        compiler_params=pltpu.CompilerParams(
            dimension_semantics=("arbitrary",),
        ),
    )(x, pe)
